# Initial kernel scaffold; baseline (speedup 1.0000x reference)
import jax
import jax.numpy as jnp
from jax.experimental import pallas as pl


def kernel(input, target):
    raise NotImplementedError("write your pallas kernel here")


# speedup vs baseline: 39.2770x; 39.2770x over previous
import jax
import jax.numpy as jnp
from jax.experimental import pallas as pl
from jax.experimental.pallas import tpu as pltpu

OHEM_RATIO = 0.2
ROWS = 64
COLS = 8192
N = ROWS * COLS
K = max(1, int(N * OHEM_RATIO))
HI_BITS = 0x7F800000
SEARCH_ITERS = 31


def _ohem_kernel(inp_ref, tgt_ref, out_ref, loss_ref):
    inp = inp_ref[...]
    tgt = tgt_ref[...]
    log_p = jnp.maximum(jnp.log(inp), -100.0)
    log_1mp = jnp.maximum(jnp.log1p(-inp), -100.0)
    loss = -(tgt * log_p + (1.0 - tgt) * log_1mp)
    loss_ref[...] = loss

    def body(_, carry):
        lo, hi = carry
        mid = lo + (hi - lo) // 2
        thr = jax.lax.bitcast_convert_type(mid, jnp.float32)
        cnt = jnp.sum((loss_ref[...] >= thr).astype(jnp.int32))
        ge_k = cnt >= K
        return jnp.where(ge_k, mid, lo), jnp.where(ge_k, hi, mid)

    lo0 = jnp.int32(0)
    hi0 = jnp.int32(HI_BITS)
    lo, _ = jax.lax.fori_loop(0, SEARCH_ITERS, body, (lo0, hi0))

    t = jax.lax.bitcast_convert_type(lo, jnp.float32)
    lv = loss_ref[...]
    gt = lv > t
    sum_gt = jnp.sum(jnp.where(gt, lv, 0.0))
    cnt_gt = jnp.sum(gt.astype(jnp.int32))
    total = sum_gt + (K - cnt_gt).astype(jnp.float32) * t
    out_ref[0, 0] = total / jnp.float32(K)


def kernel(input, target):
    out = pl.pallas_call(
        _ohem_kernel,
        out_shape=jax.ShapeDtypeStruct((1, 1), jnp.float32),
        out_specs=pl.BlockSpec(memory_space=pltpu.SMEM),
        scratch_shapes=[pltpu.VMEM((ROWS, COLS), jnp.float32)],
    )(input, target)
    return out[0, 0]


# speedup vs baseline: 57.0030x; 1.4513x over previous
import jax
import jax.numpy as jnp
from jax.experimental import pallas as pl
from jax.experimental.pallas import tpu as pltpu

OHEM_RATIO = 0.2
ROWS = 64
COLS = 8192
N = ROWS * COLS
K = max(1, int(N * OHEM_RATIO))
HI_BITS = 0x42C80001
SEARCH_ITERS = 18


def _ohem_kernel(inp_ref, tgt_ref, out_ref, loss_ref):
    inp = inp_ref[...]
    tgt = tgt_ref[...]
    log_p = jnp.maximum(jnp.log(inp), -100.0)
    log_1mp = jnp.maximum(jnp.log1p(-inp), -100.0)
    loss = -(tgt * log_p + (1.0 - tgt) * log_1mp)
    loss_ref[...] = loss

    def body(_, carry):
        lo, hi = carry
        mid = lo + (hi - lo) // 2
        thr = jax.lax.bitcast_convert_type(mid, jnp.float32)
        cnt = jnp.sum((loss_ref[...] >= thr).astype(jnp.int32))
        ge_k = cnt >= K
        return jnp.where(ge_k, mid, lo), jnp.where(ge_k, hi, mid)

    lo0 = jnp.int32(0)
    hi0 = jnp.int32(HI_BITS)
    lo, hi = jax.lax.fori_loop(0, SEARCH_ITERS, body, (lo0, hi0))

    f_lo = jax.lax.bitcast_convert_type(lo, jnp.float32)
    f_hi = jax.lax.bitcast_convert_type(hi, jnp.float32)
    lv = loss_ref[...]
    ge_hi = lv >= f_hi
    in_br = jnp.logical_and(lv >= f_lo, jnp.logical_not(ge_hi))
    sum_hi = jnp.sum(jnp.where(ge_hi, lv, 0.0))
    cnt_hi = jnp.sum(ge_hi.astype(jnp.int32))
    sum_br = jnp.sum(jnp.where(in_br, lv, 0.0))
    cnt_br = jnp.sum(in_br.astype(jnp.int32))
    avg_br = sum_br / cnt_br.astype(jnp.float32)
    total = sum_hi + (K - cnt_hi).astype(jnp.float32) * avg_br
    out_ref[0, 0] = total / jnp.float32(K)


def kernel(input, target):
    out = pl.pallas_call(
        _ohem_kernel,
        out_shape=jax.ShapeDtypeStruct((1, 1), jnp.float32),
        out_specs=pl.BlockSpec(memory_space=pltpu.SMEM),
        scratch_shapes=[pltpu.VMEM((ROWS, COLS), jnp.float32)],
    )(input, target)
    return out[0, 0]
